# Initial kernel scaffold; baseline (speedup 1.0000x reference)
import jax
import jax.numpy as jnp
from jax.experimental import pallas as pl


def kernel(x, edge_index, W):
    raise NotImplementedError("write your pallas kernel here")


# speedup vs baseline: 11.6925x; 11.6925x over previous
import functools

import numpy as np

import jax
import jax.numpy as jnp
from jax import lax
from jax.experimental import pallas as pl
from jax.experimental.pallas import tpu as pltpu
from jax.experimental.pallas import tpu_sc as plsc

_NCORE = 2
_NSUB = 16
_NW = _NCORE * _NSUB
_CHUNK = 128
_I0 = np.int32(0)


def _sc_segment_sum(src, dst, table, zeros, ones, n_pad, n_chunks, gather):
    c_dim = zeros.shape[1]
    rows_per_sub = n_pad // _NSUB
    mesh = plsc.VectorSubcoreMesh(core_axis_name="c", subcore_axis_name="s")

    scratch = [
        pltpu.VMEM((_CHUNK,), jnp.int32),
        pltpu.VMEM((_CHUNK,), jnp.int32),
        pltpu.VMEM((_CHUNK, c_dim), jnp.float32),
        pltpu.VMEM_SHARED((n_pad, c_dim), jnp.float32),
        pltpu.VMEM_SHARED((n_pad, c_dim), jnp.float32),
        pltpu.SemaphoreType.DMA,
    ]

    @functools.partial(
        pl.kernel,
        out_type=jax.ShapeDtypeStruct((_NCORE, n_pad, c_dim), jnp.float32),
        mesh=mesh,
        scratch_types=scratch,
    )
    def body(src_hbm, dst_hbm, table_hbm, zeros_hbm, ones_hbm, out_hbm,
             src_v, dst_v, rows_v, acc_sh, tab_sh, sem):
        cid = lax.axis_index("c")
        sid = lax.axis_index("s")
        wid = sid * jnp.int32(_NCORE) + cid
        r0 = sid * jnp.int32(rows_per_sub)
        pltpu.sync_copy(zeros_hbm.at[pl.ds(r0, rows_per_sub)],
                        acc_sh.at[pl.ds(r0, rows_per_sub)])
        if gather:
            pltpu.sync_copy(table_hbm.at[pl.ds(r0, rows_per_sub)],
                            tab_sh.at[pl.ds(r0, rows_per_sub)])
        else:
            pltpu.sync_copy(ones_hbm, rows_v)
        plsc.subcore_barrier()
        base = wid * jnp.int32(n_chunks * _CHUNK)

        def step(j, carry):
            off = base + j * jnp.int32(_CHUNK)
            pltpu.sync_copy(dst_hbm.at[pl.ds(off, _CHUNK)], dst_v)
            if gather:
                pltpu.sync_copy(src_hbm.at[pl.ds(off, _CHUNK)], src_v)
                pltpu.async_copy(tab_sh.at[src_v], rows_v, sem).wait()
            pltpu.sync_copy(rows_v, acc_sh.at[dst_v], add=True)
            return carry

        lax.fori_loop(jnp.int32(0), jnp.int32(n_chunks), step, jnp.int32(0))
        plsc.subcore_barrier()
        pltpu.sync_copy(acc_sh.at[pl.ds(r0, rows_per_sub)],
                        out_hbm.at[cid, pl.ds(r0, rows_per_sub)])

    return body(src, dst, table, zeros, ones)


def _encoder_tc(x, w, deg_partials):
    n, d = x.shape
    c = w.shape[1]
    bm = 1000

    def body(x_ref, w_ref, dp_ref, y_ref, norm_ref):
        xw = lax.dot_general(x_ref[...], w_ref[...],
                             (((1,), (0,)), ((), ())),
                             preferred_element_type=jnp.float32)
        cnt = dp_ref[0] + dp_ref[1]
        norm = lax.rsqrt(cnt + 1.0)
        norm_ref[...] = norm
        y_ref[...] = norm * xw

    return pl.pallas_call(
        body,
        grid=(n // bm,),
        in_specs=[
            pl.BlockSpec((bm, d), lambda i: (i, _I0)),
            pl.BlockSpec((d, c), lambda i: (_I0, _I0)),
            pl.BlockSpec((2, bm, c), lambda i: (_I0, i, _I0)),
        ],
        out_specs=[
            pl.BlockSpec((bm, c), lambda i: (i, _I0)),
            pl.BlockSpec((bm, c), lambda i: (i, _I0)),
        ],
        out_shape=[
            jax.ShapeDtypeStruct((n, c), jnp.float32),
            jax.ShapeDtypeStruct((n, c), jnp.float32),
        ],
    )(x, w, deg_partials)


def _decoder_tc(s_partials, y, norm):
    n, c = y.shape
    n_pad = s_partials.shape[1]
    bm = 200

    def body(s_ref, y_ref, norm_ref, out_ref, h_ref):
        i = pl.program_id(0)

        @pl.when(i == 0)
        def _():
            s = s_ref[0, :n, :] + s_ref[1, :n, :]
            h_ref[...] = jnp.maximum(norm_ref[...] * (s + y_ref[...]), 0.0)

        hm = h_ref[pl.ds(i * bm, bm), :]
        z = lax.dot_general(hm, h_ref[...], (((1,), (1,)), ((), ())),
                            preferred_element_type=jnp.float32)
        out_ref[...] = 0.5 * jnp.tanh(0.5 * z) + 0.5

    return pl.pallas_call(
        body,
        grid=(n // bm,),
        in_specs=[
            pl.BlockSpec((2, n_pad, c), lambda i: (_I0, _I0, _I0)),
            pl.BlockSpec((n, c), lambda i: (_I0, _I0)),
            pl.BlockSpec((n, c), lambda i: (_I0, _I0)),
        ],
        out_specs=pl.BlockSpec((bm, n), lambda i: (i, _I0)),
        out_shape=jax.ShapeDtypeStruct((n, n), jnp.float32),
        scratch_shapes=[pltpu.VMEM((n, c), jnp.float32)],
    )(s_partials, y, norm)


def kernel(x, edge_index, W):
    n, _ = x.shape
    c = W.shape[1]
    e = edge_index.shape[1]
    ei = edge_index.astype(jnp.int32)

    e_per_w = -(-e // _NW)
    n_chunks = -(-e_per_w // _CHUNK)
    e_pad = _NW * n_chunks * _CHUNK
    pad = e_pad - e
    src = jnp.concatenate([ei[0], jnp.full((pad,), n, jnp.int32)])
    dst = jnp.concatenate([ei[1], jnp.full((pad,), n, jnp.int32)])

    n_pad = -(-(n + 1) // (_NSUB * 8)) * (_NSUB * 8)
    zeros = jnp.zeros((n_pad, c), jnp.float32)
    ones = jnp.ones((_CHUNK, c), jnp.float32)
    dummy_table = zeros

    deg_part = _sc_segment_sum(src, dst, dummy_table, zeros, ones,
                               n_pad, n_chunks, gather=False)
    y, norm = _encoder_tc(x.astype(jnp.float32), W.astype(jnp.float32),
                          deg_part)
    y_pad = jnp.concatenate([y, jnp.zeros((n_pad - n, c), jnp.float32)])
    s_part = _sc_segment_sum(src, dst, y_pad, zeros, ones,
                             n_pad, n_chunks, gather=True)
    return _decoder_tc(s_part, y, norm).astype(jnp.float64)
